# Initial kernel scaffold; baseline (speedup 1.0000x reference)
import jax
import jax.numpy as jnp
from jax.experimental import pallas as pl


def kernel(x, edge_index, edge_weight, W):
    raise NotImplementedError("write your pallas kernel here")


# speedup vs baseline: 4.4873x; 4.4873x over previous
import functools

import jax
import jax.numpy as jnp
from jax import lax
from jax.experimental import pallas as pl
from jax.experimental.pallas import tpu as pltpu
from jax.experimental.pallas import tpu_sc as plsc

NC = 2
NS = 16
L = 16
NW = NC * NS


def _matmul(x, W):
    n, d_in = x.shape
    d_out = W.shape[1]
    bm = 1000

    def body(x_ref, w_ref, o_ref):
        o_ref[...] = jnp.dot(x_ref[...], w_ref[...],
                             preferred_element_type=jnp.float32)

    return pl.pallas_call(
        body,
        grid=(n // bm,),
        in_specs=[
            pl.BlockSpec((bm, d_in), lambda i: (i, 0)),
            pl.BlockSpec((d_in, d_out), lambda i: (0, 0)),
        ],
        out_specs=pl.BlockSpec((bm, d_out), lambda i: (i, 0)),
        out_shape=jax.ShapeDtypeStruct((n, d_out), jnp.float32),
    )(x, W)


def _sc_scatter(support, src, dst, ew):
    n, d = support.shape
    e = src.shape[0]
    epw = e // NW
    c = 80
    nchunk = epw // c
    rblk = 80
    nblk = n // rblk
    dvec = d // L

    mesh = plsc.VectorSubcoreMesh(core_axis_name="c", subcore_axis_name="s")

    @functools.partial(
        pl.kernel,
        out_type=jax.ShapeDtypeStruct((NC, n, d), jnp.float32),
        mesh=mesh,
        scratch_types=[
            pltpu.VMEM((c,), jnp.int32),
            pltpu.VMEM((c,), jnp.int32),
            pltpu.VMEM((c,), jnp.float32),
            pltpu.VMEM((c, d), jnp.float32),
            pltpu.VMEM_SHARED((n, d), jnp.float32),
            pltpu.SemaphoreType.DMA,
        ],
    )
    def sc_body(sup_hbm, src_hbm, dst_hbm, ew_hbm, out_hbm,
                src_v, dst_v, ew_v, rows_v, acc, sem):
        cid = lax.axis_index("c")
        sid = lax.axis_index("s")
        wid = cid * NS + sid
        ebase = wid * epw
        nb = (nblk - 1 - sid) // NS + 1

        zv = jnp.zeros((L,), jnp.float32)

        def zrow(i, _):
            for j in range(dvec):
                rows_v[i, pl.ds(j * L, L)] = zv
            return 0

        lax.fori_loop(0, c, zrow, 0)

        def zblk(k, _):
            r = pl.multiple_of((sid + k * NS) * rblk, 8)
            pltpu.sync_copy(rows_v, acc.at[pl.ds(r, rblk)])
            return 0

        lax.fori_loop(0, nb, zblk, 0)
        plsc.subcore_barrier()

        def chunk(i, _):
            eb = pl.multiple_of(ebase + i * c, 8)
            pltpu.sync_copy(src_hbm.at[pl.ds(eb, c)], src_v)
            pltpu.sync_copy(dst_hbm.at[pl.ds(eb, c)], dst_v)
            pltpu.sync_copy(ew_hbm.at[pl.ds(eb, c)], ew_v)
            pltpu.async_copy(sup_hbm.at[src_v], rows_v, sem).wait()

            def scale_group(g, _):
                ew16 = ew_v[pl.ds(g * L, L)]

                def scale_edge(t, _):
                    wb = ew16.at[jnp.full((L,), t, jnp.int32)].get(
                        mode="promise_in_bounds")
                    ei = g * L + t
                    for j in range(dvec):
                        sl = pl.ds(j * L, L)
                        rows_v[ei, sl] = rows_v[ei, sl] * wb
                    return 0

                lax.fori_loop(0, L, scale_edge, 0)
                return 0

            lax.fori_loop(0, c // L, scale_group, 0)
            pltpu.sync_copy(rows_v, acc.at[dst_v], add=True)
            return 0

        lax.fori_loop(0, nchunk, chunk, 0)
        plsc.subcore_barrier()

        def wblk(k, _):
            r = pl.multiple_of((sid + k * NS) * rblk, 8)
            pltpu.sync_copy(acc.at[pl.ds(r, rblk)],
                            out_hbm.at[cid].at[pl.ds(r, rblk)])
            return 0

        lax.fori_loop(0, nb, wblk, 0)

    return sc_body(support, src, dst, ew)


def _combine(partials):
    _, n, d = partials.shape
    bm = 1000

    def body(p_ref, o_ref):
        o_ref[...] = jnp.maximum(p_ref[0] + p_ref[1], 0.0)

    return pl.pallas_call(
        body,
        grid=(n // bm,),
        in_specs=[pl.BlockSpec((NC, bm, d), lambda i: (0, i, 0))],
        out_specs=pl.BlockSpec((bm, d), lambda i: (i, 0)),
        out_shape=jax.ShapeDtypeStruct((n, d), jnp.float32),
    )(partials)


@jax.jit
def kernel(x, edge_index, edge_weight, W):
    support = _matmul(x, W)
    partials = _sc_scatter(support, edge_index[0], edge_index[1], edge_weight)
    return _combine(partials)


# speedup vs baseline: 9.3453x; 2.0826x over previous
import functools

import jax
import jax.numpy as jnp
from jax import lax
from jax.experimental import pallas as pl
from jax.experimental.pallas import tpu as pltpu
from jax.experimental.pallas import tpu_sc as plsc

NC = 2
NS = 16
L = 16
NW = NC * NS


def _matmul(x, W):
    n, d_in = x.shape
    d_out = W.shape[1]
    bm = 1000

    def body(x_ref, w_ref, o_ref):
        o_ref[...] = jnp.dot(x_ref[...], w_ref[...],
                             preferred_element_type=jnp.float32)

    return pl.pallas_call(
        body,
        grid=(n // bm,),
        in_specs=[
            pl.BlockSpec((bm, d_in), lambda i: (i, 0)),
            pl.BlockSpec((d_in, d_out), lambda i: (0, 0)),
        ],
        out_specs=pl.BlockSpec((bm, d_out), lambda i: (i, 0)),
        out_shape=jax.ShapeDtypeStruct((n, d_out), jnp.float32),
    )(x, W)


def _sc_scatter(support, src, dst, ew):
    n, d = support.shape
    e = ew.shape[0]
    epw = e // NW
    c = 80
    sck = 25
    nsc = epw // (sck * c)
    rblk = 80
    nblk = n // rblk
    dvec = d // L

    src3 = src.reshape(NW * nsc, sck, c)
    dst3 = dst.reshape(NW * nsc, sck, c)
    ew2 = ew.reshape(NW * nsc, sck * c)

    mesh = plsc.VectorSubcoreMesh(core_axis_name="c", subcore_axis_name="s")

    @functools.partial(
        pl.kernel,
        out_type=jax.ShapeDtypeStruct((NC, n, d), jnp.float32),
        mesh=mesh,
        scratch_types=[
            pltpu.VMEM((sck, c), jnp.int32),
            pltpu.VMEM((sck, c), jnp.int32),
            pltpu.VMEM((sck * c,), jnp.float32),
            pltpu.VMEM((c, d), jnp.float32),
            pltpu.VMEM((c, d), jnp.float32),
            pltpu.VMEM_SHARED((n, d), jnp.float32),
            pltpu.SemaphoreType.DMA,
            pltpu.SemaphoreType.DMA,
        ],
    )
    def sc_body(sup_hbm, src_hbm, dst_hbm, ew_hbm, out_hbm,
                src_v, dst_v, ew_v, rows0, rows1, acc, sem0, sem1):
        cid = lax.axis_index("c")
        sid = lax.axis_index("s")
        wid = cid * NS + sid
        nb = (nblk - 1 - sid) // NS + 1

        zv = jnp.zeros((L,), jnp.float32)

        def zrow(i, _):
            for j in range(dvec):
                rows0[i, pl.ds(j * L, L)] = zv
            return 0

        lax.fori_loop(0, c, zrow, 0)

        def zblk(k, _):
            r = pl.multiple_of((sid + k * NS) * rblk, 8)
            pltpu.sync_copy(rows0, acc.at[pl.ds(r, rblk)])
            return 0

        lax.fori_loop(0, nb, zblk, 0)
        plsc.subcore_barrier()

        def scale(buf, i):
            def sgroup(g, _):
                ew16 = ew_v[pl.ds(i * c + g * L, L)]
                for t in range(L):
                    wb = ew16.at[jnp.full((L,), t, jnp.int32)].get(
                        mode="promise_in_bounds")
                    row = g * L + t
                    for k in range(dvec):
                        sl = pl.ds(k * L, L)
                        buf[row, sl] = buf[row, sl] * wb
                return 0

            lax.fori_loop(0, c // L, sgroup, 0)

        def gather(i, buf, sem):
            pltpu.async_copy(sup_hbm.at[src_v.at[i]], buf, sem)

        def gwait(i, buf, sem):
            pltpu.make_async_copy(sup_hbm.at[src_v.at[i]], buf, sem).wait()

        def scatter(buf, i):
            pltpu.sync_copy(buf, acc.at[dst_v.at[i]], add=True)

        def superchunk(s, _):
            sc_row = wid * nsc + s
            pltpu.sync_copy(src_hbm.at[sc_row], src_v)
            pltpu.sync_copy(dst_hbm.at[sc_row], dst_v)
            pltpu.sync_copy(ew_hbm.at[sc_row], ew_v)

            gather(0, rows0, sem0)

            def pair(j, _):
                i0 = j * 2
                gwait(i0, rows0, sem0)
                gather(i0 + 1, rows1, sem1)
                scale(rows0, i0)
                scatter(rows0, i0)
                gwait(i0 + 1, rows1, sem1)
                gather(i0 + 2, rows0, sem0)
                scale(rows1, i0 + 1)
                scatter(rows1, i0 + 1)
                return 0

            lax.fori_loop(0, (sck - 1) // 2, pair, 0)
            last = sck - 1
            gwait(last, rows0, sem0)
            scale(rows0, last)
            scatter(rows0, last)
            return 0

        lax.fori_loop(0, nsc, superchunk, 0)
        plsc.subcore_barrier()

        def wblk(k, _):
            r = pl.multiple_of((sid + k * NS) * rblk, 8)
            pltpu.sync_copy(acc.at[pl.ds(r, rblk)],
                            out_hbm.at[cid].at[pl.ds(r, rblk)])
            return 0

        lax.fori_loop(0, nb, wblk, 0)

    return sc_body(support, src3, dst3, ew2)


def _combine(partials):
    _, n, d = partials.shape
    bm = 1000

    def body(p_ref, o_ref):
        o_ref[...] = jnp.maximum(p_ref[0] + p_ref[1], 0.0)

    return pl.pallas_call(
        body,
        grid=(n // bm,),
        in_specs=[pl.BlockSpec((NC, bm, d), lambda i: (0, i, 0))],
        out_specs=pl.BlockSpec((bm, d), lambda i: (i, 0)),
        out_shape=jax.ShapeDtypeStruct((n, d), jnp.float32),
    )(partials)


@jax.jit
def kernel(x, edge_index, edge_weight, W):
    support = _matmul(x, W)
    partials = _sc_scatter(support, edge_index[0], edge_index[1], edge_weight)
    return _combine(partials)


# speedup vs baseline: 10.9463x; 1.1713x over previous
import functools

import jax
import jax.numpy as jnp
from jax import lax
from jax.experimental import pallas as pl
from jax.experimental.pallas import tpu as pltpu
from jax.experimental.pallas import tpu_sc as plsc

NC = 2
NS = 16
L = 16
NW = NC * NS


def _matmul(x, W):
    n, d_in = x.shape
    d_out = W.shape[1]
    bm = 1000

    def body(x_ref, w_ref, o_ref):
        o_ref[...] = jnp.dot(x_ref[...], w_ref[...],
                             preferred_element_type=jnp.float32)

    return pl.pallas_call(
        body,
        grid=(n // bm,),
        in_specs=[
            pl.BlockSpec((bm, d_in), lambda i: (i, 0)),
            pl.BlockSpec((d_in, d_out), lambda i: (0, 0)),
        ],
        out_specs=pl.BlockSpec((bm, d_out), lambda i: (i, 0)),
        out_shape=jax.ShapeDtypeStruct((n, d_out), jnp.float32),
    )(x, W)


def _sc_scatter(support, src, dst, ew):
    n, d = support.shape
    e = ew.shape[0]
    epw = e // NW
    c = 80
    sck = 25
    nsc = epw // (sck * c)
    rblk = 80
    nblk = n // rblk
    dvec = d // L

    src3 = src.reshape(NW * nsc, sck, c)
    dst3 = dst.reshape(NW * nsc, sck, c)
    ew2 = ew.reshape(NW * nsc, sck * c)

    mesh = plsc.VectorSubcoreMesh(core_axis_name="c", subcore_axis_name="s")

    @functools.partial(
        pl.kernel,
        out_type=jax.ShapeDtypeStruct((NC, n, d), jnp.float32),
        mesh=mesh,
        scratch_types=[
            pltpu.VMEM((sck, c), jnp.int32),
            pltpu.VMEM((sck, c), jnp.int32),
            pltpu.VMEM((sck * c,), jnp.float32),
            pltpu.VMEM((c, d), jnp.float32),
            pltpu.VMEM((c, d), jnp.float32),
            pltpu.VMEM((c, d), jnp.float32),
            pltpu.VMEM_SHARED((n, d), jnp.float32),
            pltpu.SemaphoreType.DMA,
            pltpu.SemaphoreType.DMA,
            pltpu.SemaphoreType.DMA,
            pltpu.SemaphoreType.DMA,
            pltpu.SemaphoreType.DMA,
            pltpu.SemaphoreType.DMA,
        ],
    )
    def sc_body(sup_hbm, src_hbm, dst_hbm, ew_hbm, out_hbm,
                src_v, dst_v, ew_v, rows0, rows1, rows2, acc,
                gsem0, gsem1, gsem2, ssem0, ssem1, ssem2):
        cid = lax.axis_index("c")
        sid = lax.axis_index("s")
        wid = cid * NS + sid
        nb = (nblk - 1 - sid) // NS + 1

        zv = jnp.zeros((L,), jnp.float32)

        def zrow(i, _):
            for j in range(dvec):
                rows0[i, pl.ds(j * L, L)] = zv
            return 0

        lax.fori_loop(0, c, zrow, 0)

        def zblk(k, _):
            r = pl.multiple_of((sid + k * NS) * rblk, 8)
            pltpu.sync_copy(rows0, acc.at[pl.ds(r, rblk)])
            return 0

        lax.fori_loop(0, nb, zblk, 0)
        plsc.subcore_barrier()

        def scale(buf, i):
            def sgroup(g, _):
                ew16 = ew_v[pl.ds(i * c + g * L, L)]
                for t in range(L):
                    wb = ew16.at[jnp.full((L,), t, jnp.int32)].get(
                        mode="promise_in_bounds")
                    row = g * L + t
                    for k in range(dvec):
                        sl = pl.ds(k * L, L)
                        buf[row, sl] = buf[row, sl] * wb
                return 0

            lax.fori_loop(0, c // L, sgroup, 0)

        bufs = ((rows0, gsem0, ssem0),
                (rows1, gsem1, ssem1),
                (rows2, gsem2, ssem2))

        def gather(i, buf, sem):
            pltpu.async_copy(sup_hbm.at[src_v.at[i]], buf, sem)

        def gwait(i, buf, sem):
            pltpu.make_async_copy(sup_hbm.at[src_v.at[i]], buf, sem).wait()

        def scatter_start(buf, i, sem):
            pltpu.async_copy(buf, acc.at[dst_v.at[i]], sem, add=True)

        def scatter_drain(buf, i, sem):
            pltpu.make_async_copy(buf, acc.at[dst_v.at[i]], sem).wait()

        def superchunk(s, _):
            sc_row = wid * nsc + s
            pltpu.sync_copy(src_hbm.at[sc_row], src_v)
            pltpu.sync_copy(dst_hbm.at[sc_row], dst_v)
            pltpu.sync_copy(ew_hbm.at[sc_row], ew_v)

            gather(0, rows0, gsem0)
            gather(1, rows1, gsem1)

            def triple(t, _):
                for p in range(3):
                    i = t * 3 + p
                    buf, gs, _ss = bufs[p]
                    nbuf, ngs, nss = bufs[(p + 2) % 3]
                    gwait(i, buf, gs)
                    if p == 0:
                        @pl.when(t >= 1)
                        def _():
                            scatter_drain(nbuf, i - 1, nss)
                    elif p < 2:
                        scatter_drain(nbuf, i - 1, nss)
                    if p == 2:
                        @pl.when(t < (sck // 3) - 1)
                        def _():
                            scatter_drain(nbuf, i - 1, nss)
                            gather(i + 2, nbuf, ngs)
                    else:
                        gather(i + 2, nbuf, ngs)
                    scale(buf, i)
                    scatter_start(buf, i, _ss)
                return 0

            lax.fori_loop(0, sck // 3, triple, 0)
            last = sck - 1
            gwait(last, rows0, gsem0)
            scale(rows0, last)
            pltpu.sync_copy(rows0, acc.at[dst_v.at[last]], add=True)
            scatter_drain(rows1, sck - 3, ssem1)
            scatter_drain(rows2, sck - 2, ssem2)
            return 0

        lax.fori_loop(0, nsc, superchunk, 0)
        plsc.subcore_barrier()

        def wblk(k, _):
            r = pl.multiple_of((sid + k * NS) * rblk, 8)
            pltpu.sync_copy(acc.at[pl.ds(r, rblk)],
                            out_hbm.at[cid].at[pl.ds(r, rblk)])
            return 0

        lax.fori_loop(0, nb, wblk, 0)

    return sc_body(support, src3, dst3, ew2)


def _combine(partials):
    _, n, d = partials.shape
    bm = 1000

    def body(p_ref, o_ref):
        o_ref[...] = jnp.maximum(p_ref[0] + p_ref[1], 0.0)

    return pl.pallas_call(
        body,
        grid=(n // bm,),
        in_specs=[pl.BlockSpec((NC, bm, d), lambda i: (0, i, 0))],
        out_specs=pl.BlockSpec((bm, d), lambda i: (i, 0)),
        out_shape=jax.ShapeDtypeStruct((n, d), jnp.float32),
    )(partials)


@jax.jit
def kernel(x, edge_index, edge_weight, W):
    support = _matmul(x, W)
    partials = _sc_scatter(support, edge_index[0], edge_index[1], edge_weight)
    return _combine(partials)


# speedup vs baseline: 11.5797x; 1.0579x over previous
import functools

import jax
import jax.numpy as jnp
import numpy as np
from jax import lax
from jax.experimental import pallas as pl
from jax.experimental.pallas import tpu as pltpu
from jax.experimental.pallas import tpu_sc as plsc

NC = 2
NS = 16
L = 16
NW = NC * NS


def _col_perm(d):
    perm = np.empty((d,), np.int32)
    for k in range(d // 32):
        for i in range(16):
            perm[2 * (16 * k + i)] = 32 * k + i
            perm[2 * (16 * k + i) + 1] = 32 * k + 16 + i
    return perm


def _matmul(x, W):
    n, d_in = x.shape
    d_out = W.shape[1]
    bm = 1000

    def body(x_ref, w_ref, o_ref):
        o_ref[...] = jnp.dot(x_ref[...], w_ref[...],
                             preferred_element_type=jnp.float32)

    return pl.pallas_call(
        body,
        grid=(n // bm,),
        in_specs=[
            pl.BlockSpec((bm, d_in), lambda i: (i, 0)),
            pl.BlockSpec((d_in, d_out), lambda i: (0, 0)),
        ],
        out_specs=pl.BlockSpec((bm, d_out), lambda i: (i, 0)),
        out_shape=jax.ShapeDtypeStruct((n, d_out), jnp.float32),
    )(x, W)


def _sc_scatter(sup, ei, ew):
    n, d = sup.shape
    e = ew.shape[0]
    epw = e // NW
    c = 80
    sck = 25
    nsc = epw // (sck * c)
    rblk = 80
    nblk = n // rblk
    dvec = d // L

    ei4 = ei.reshape(2, NW * nsc, sck, c)
    ew2 = ew.reshape(NW * nsc, sck * c)

    mesh = plsc.VectorSubcoreMesh(core_axis_name="c", subcore_axis_name="s")

    @functools.partial(
        pl.kernel,
        out_type=jax.ShapeDtypeStruct((NC, n, d), jnp.float32),
        mesh=mesh,
        scratch_types=[
            pltpu.VMEM((sck, c), jnp.int32),
            pltpu.VMEM((sck, c), jnp.int32),
            pltpu.VMEM((sck * c,), jnp.float32),
            pltpu.VMEM((c, d), jnp.float32),
            pltpu.VMEM((c, d), jnp.float32),
            pltpu.VMEM((c, d), jnp.float32),
            pltpu.VMEM_SHARED((n, d), jnp.float32),
            pltpu.SemaphoreType.DMA,
            pltpu.SemaphoreType.DMA,
            pltpu.SemaphoreType.DMA,
            pltpu.SemaphoreType.DMA,
            pltpu.SemaphoreType.DMA,
            pltpu.SemaphoreType.DMA,
        ],
    )
    def sc_body(sup_hbm, ei_hbm, ew_hbm, out_hbm,
                src_v, dst_v, ew_v, rb0, rb1, rb2, acc,
                gsem0, gsem1, gsem2, ssem0, ssem1, ssem2):
        cid = lax.axis_index("c")
        sid = lax.axis_index("s")
        wid = cid * NS + sid
        nb = (nblk - 1 - sid) // NS + 1

        zv = jnp.zeros((L,), jnp.float32)

        def zrow(i, _):
            for j in range(dvec):
                rb0[i, pl.ds(j * L, L)] = zv
            return 0

        lax.fori_loop(0, c, zrow, 0)

        def zblk(k, _):
            r = pl.multiple_of((sid + k * NS) * rblk, 8)
            pltpu.sync_copy(rb0, acc.at[pl.ds(r, rblk)])
            return 0

        lax.fori_loop(0, nb, zblk, 0)
        plsc.subcore_barrier()

        bufs = ((rb0, gsem0, ssem0), (rb1, gsem1, ssem1), (rb2, gsem2, ssem2))

        def scale(rb, i):
            def sgroup(g, _):
                ew16 = ew_v[pl.ds(i * c + g * L, L)]
                for t in range(L):
                    wb = ew16.at[jnp.full((L,), t, jnp.int32)].get(
                        mode="promise_in_bounds")
                    row = g * L + t
                    for k in range(dvec):
                        sl = pl.ds(k * L, L)
                        rb[row, sl] = rb[row, sl] * wb
                return 0

            lax.fori_loop(0, c // L, sgroup, 0)

        def gather(i, rb, sem):
            pltpu.async_copy(sup_hbm.at[src_v.at[i]], rb, sem)

        def gwait(i, rb, sem):
            pltpu.make_async_copy(sup_hbm.at[src_v.at[i]], rb, sem).wait()

        def scatter_start(rb, i, sem):
            pltpu.async_copy(rb, acc.at[dst_v.at[i]], sem, add=True)

        def scatter_drain(rb, i, sem):
            pltpu.make_async_copy(rb, acc.at[dst_v.at[i]], sem).wait()

        def superchunk(s, _):
            sc_row = wid * nsc + s
            pltpu.sync_copy(ei_hbm.at[0, sc_row], src_v)
            pltpu.sync_copy(ei_hbm.at[1, sc_row], dst_v)
            pltpu.sync_copy(ew_hbm.at[sc_row], ew_v)

            gather(0, rb0, gsem0)
            gather(1, rb1, gsem1)

            def triple(t, _):
                for p in range(3):
                    i = t * 3 + p
                    rb, gs, ss = bufs[p]
                    nrb, ngs, nss = bufs[(p + 2) % 3]
                    gwait(i, rb, gs)
                    if p == 0:
                        @pl.when(t >= 1)
                        def _():
                            scatter_drain(nrb, i - 1, nss)
                    elif p < 2:
                        scatter_drain(nrb, i - 1, nss)
                    if p == 2:
                        @pl.when(t < (sck // 3) - 1)
                        def _():
                            scatter_drain(nrb, i - 1, nss)
                            gather(i + 2, nrb, ngs)
                    else:
                        gather(i + 2, nrb, ngs)
                    scale(rb, i)
                    scatter_start(rb, i, ss)
                return 0

            lax.fori_loop(0, sck // 3, triple, 0)
            last = sck - 1
            gwait(last, rb0, gsem0)
            scale(rb0, last)
            pltpu.sync_copy(rb0, acc.at[dst_v.at[last]], add=True)
            scatter_drain(rb1, sck - 3, ssem1)
            scatter_drain(rb2, sck - 2, ssem2)
            return 0

        lax.fori_loop(0, nsc, superchunk, 0)
        plsc.subcore_barrier()

        def wblk(k, _):
            r = pl.multiple_of((sid + k * NS) * rblk, 8)
            pltpu.sync_copy(acc.at[pl.ds(r, rblk)],
                            out_hbm.at[cid].at[pl.ds(r, rblk)])
            return 0

        lax.fori_loop(0, nb, wblk, 0)

    return sc_body(sup, ei4, ew2)


def _combine(partials):
    _, n, d = partials.shape
    bm = 1000

    def body(p_ref, o_ref):
        o_ref[...] = jnp.maximum(p_ref[0] + p_ref[1], 0.0)

    return pl.pallas_call(
        body,
        grid=(n // bm,),
        in_specs=[pl.BlockSpec((NC, bm, d), lambda i: (0, i, 0))],
        out_specs=pl.BlockSpec((bm, d), lambda i: (i, 0)),
        out_shape=jax.ShapeDtypeStruct((n, d), jnp.float32),
    )(partials)


@jax.jit
def kernel(x, edge_index, edge_weight, W):
    support = _matmul(x, W)
    partials = _sc_scatter(support, edge_index, edge_weight)
    return _combine(partials)
